# Initial kernel scaffold; baseline (speedup 1.0000x reference)
import jax
import jax.numpy as jnp
from jax.experimental import pallas as pl


def kernel(nodes, neigh, feat, lap, W_in, Wq, Wk, Wv, Wo, W1, W2, W_dense, b_dense):
    raise NotImplementedError("write your pallas kernel here")


# speedup vs baseline: 1.6628x; 1.6628x over previous
import functools

import jax
import jax.numpy as jnp
import numpy as np
from jax import lax
from jax.experimental import pallas as pl
from jax.experimental.pallas import tpu as pltpu
from jax.experimental.pallas import tpu_sc as plsc

N = 100000
D = 128
DL = 16
S = 16
B = 8192
EMB = 128
H = 4
L = 2
FF = 256
C = 40
T = S + 1
BT = B * T
DH = EMB // H

BB = 128
R = BB * T
GS = 16
RG = GS * T
NG = BB // GS
NBLK = B // BB

_SC_MESH = functools.partial(
    plsc.VectorSubcoreMesh, core_axis_name="c", subcore_axis_name="s"
)


def _sc_gather_nbrows(neigh_p, rows2d):
    W = 256

    @functools.partial(
        pl.kernel,
        out_type=jax.ShapeDtypeStruct((B, 128), jnp.int32),
        mesh=_SC_MESH(),
    )
    def k(tab_hbm, i_hbm, o_hbm):
        def body(i_vmem, o_vmem):
            pltpu.sync_copy(tab_hbm.at[i_vmem.at[0]], o_vmem)

        pltpu.emit_pipeline(
            body,
            grid=(B // W,),
            in_specs=[pl.BlockSpec((1, W), lambda i: (0, i))],
            out_specs=[pl.BlockSpec((W, 128), lambda i: (i, 0))],
            core_axis_name=("c", "s"),
            dimension_semantics=(pltpu.PARALLEL,),
        )(i_hbm, o_hbm)

    return k(neigh_p, rows2d)


def _sc_gather_xin(xin, tok2d):
    W = 256

    @functools.partial(
        pl.kernel,
        out_type=jax.ShapeDtypeStruct((BT, EMB), jnp.float32),
        mesh=_SC_MESH(),
    )
    def k(tab_hbm, i_hbm, o_hbm):
        def body(i_vmem, o_vmem):
            pltpu.sync_copy(tab_hbm.at[i_vmem.at[0]], o_vmem)

        pltpu.emit_pipeline(
            body,
            grid=(BT // W,),
            in_specs=[pl.BlockSpec((1, W), lambda i: (0, i))],
            out_specs=[pl.BlockSpec((W, EMB), lambda i: (i, 0))],
            core_axis_name=("c", "s"),
            dimension_semantics=(pltpu.PARALLEL,),
        )(i_hbm, o_hbm)

    return k(xin, tok2d)


def _proj_body(feat_ref, lap_ref, wif_ref, wil_ref, o_ref):
    o_ref[...] = (
        jnp.dot(feat_ref[...], wif_ref[...],
                preferred_element_type=jnp.float32)
        + jnp.dot(lap_ref[...], wil_ref[...],
                  preferred_element_type=jnp.float32))


def _proj_kernel(feat, lap, w_in):
    blk = 8192
    nb = (N + 1 + blk - 1) // blk
    return pl.pallas_call(
        _proj_body,
        grid=(nb,),
        in_specs=[
            pl.BlockSpec((blk, D), lambda i: (i, 0)),
            pl.BlockSpec((blk, DL), lambda i: (i, 0)),
            pl.BlockSpec((D, EMB), lambda i: (0, 0)),
            pl.BlockSpec((DL, EMB), lambda i: (0, 0)),
        ],
        out_specs=pl.BlockSpec((blk, EMB), lambda i: (i, 0)),
        out_shape=jax.ShapeDtypeStruct((N + 1, EMB), jnp.float32),
        compiler_params=pltpu.CompilerParams(
            dimension_semantics=("parallel",)),
    )(feat, lap, w_in[:D], w_in[D:])


def _ln(z):
    m = jnp.mean(z, axis=-1, keepdims=True)
    v = jnp.mean((z - m) * (z - m), axis=-1, keepdims=True)
    return (z - m) / jnp.sqrt(v + 1e-5)


def _softmax(s):
    m = jnp.max(s, axis=-1, keepdims=True)
    e = jnp.exp(s - m)
    return e / jnp.sum(e, axis=-1, keepdims=True)


def _tc_body(px_ref, wq_ref, wk_ref, wv_ref,
             wo_ref, w1_ref, w2_ref, wd_ref, bd_ref, out_ref,
             q_ref, k_ref, v_ref, o_ref, qs_ref, o2_ref):
    f32 = jnp.float32
    scale = f32(1.0 / np.sqrt(DH))

    x = px_ref[...]

    r1 = lax.broadcasted_iota(jnp.int32, (RG, RG), 0)
    c1 = lax.broadcasted_iota(jnp.int32, (RG, RG), 1)
    mask1 = jnp.where((r1 // T) == (c1 // T), f32(0.0), f32(-1e30))
    r2 = lax.broadcasted_iota(jnp.int32, (GS, RG), 0)
    c2 = lax.broadcasted_iota(jnp.int32, (GS, RG), 1)
    mask2 = jnp.where((c2 // T) == r2, f32(0.0), f32(-1e30))

    q_ref[...] = jnp.dot(x, wq_ref[0], preferred_element_type=f32)
    k_ref[...] = jnp.dot(x, wk_ref[0], preferred_element_type=f32)
    v_ref[...] = jnp.dot(x, wv_ref[0], preferred_element_type=f32)

    def grp0(g, carry):
        base = g * RG
        for h in range(H):
            cs = slice(h * DH, (h + 1) * DH)
            qh = q_ref[pl.ds(base, RG), cs]
            kh = k_ref[pl.ds(base, RG), cs]
            vh = v_ref[pl.ds(base, RG), cs]
            s = lax.dot_general(qh, kh, (((1,), (1,)), ((), ())),
                                preferred_element_type=f32) * scale + mask1
            p = _softmax(s)
            o_ref[pl.ds(base, RG), cs] = jnp.dot(
                p, vh, preferred_element_type=f32)
        return carry

    lax.fori_loop(0, NG, grp0, 0)

    x = _ln(x + jnp.dot(o_ref[...], wo_ref[0], preferred_element_type=f32))
    ff = jnp.dot(jax.nn.relu(jnp.dot(x, w1_ref[0],
                                     preferred_element_type=f32)),
                 w2_ref[0], preferred_element_type=f32)
    x = _ln(x + ff)

    rs = lax.broadcasted_iota(jnp.int32, (BB, R), 0)
    cc = lax.broadcasted_iota(jnp.int32, (BB, R), 1)
    sel = jnp.where(cc == rs * T, f32(1.0), f32(0.0))
    xs = jnp.dot(sel, x, preferred_element_type=f32)

    qs_ref[...] = jnp.dot(xs, wq_ref[1], preferred_element_type=f32)
    k_ref[...] = jnp.dot(x, wk_ref[1], preferred_element_type=f32)
    v_ref[...] = jnp.dot(x, wv_ref[1], preferred_element_type=f32)

    def grp1(g, carry):
        base = g * RG
        sbase = g * GS
        for h in range(H):
            cs = slice(h * DH, (h + 1) * DH)
            qh = qs_ref[pl.ds(sbase, GS), cs]
            kh = k_ref[pl.ds(base, RG), cs]
            vh = v_ref[pl.ds(base, RG), cs]
            s = lax.dot_general(qh, kh, (((1,), (1,)), ((), ())),
                                preferred_element_type=f32) * scale + mask2
            p = _softmax(s)
            o2_ref[pl.ds(sbase, GS), cs] = jnp.dot(
                p, vh, preferred_element_type=f32)
        return carry

    lax.fori_loop(0, NG, grp1, 0)

    xs = _ln(xs + jnp.dot(o2_ref[...], wo_ref[1], preferred_element_type=f32))
    ff2 = jnp.dot(jax.nn.relu(jnp.dot(xs, w1_ref[1],
                                      preferred_element_type=f32)),
                  w2_ref[1], preferred_element_type=f32)
    xs = _ln(xs + ff2)

    out_ref[...] = (jnp.dot(xs, wd_ref[...], preferred_element_type=f32)
                    + bd_ref[...])


def _tc_transformer(px, wq, wk, wv, wo, w1, w2, wd, bd):
    f32 = jnp.float32
    bd2 = bd.reshape(1, C)

    return pl.pallas_call(
        _tc_body,
        grid=(NBLK,),
        in_specs=[
            pl.BlockSpec((R, EMB), lambda i: (i, 0)),
            pl.BlockSpec((L, EMB, EMB), lambda i: (0, 0, 0)),
            pl.BlockSpec((L, EMB, EMB), lambda i: (0, 0, 0)),
            pl.BlockSpec((L, EMB, EMB), lambda i: (0, 0, 0)),
            pl.BlockSpec((L, EMB, EMB), lambda i: (0, 0, 0)),
            pl.BlockSpec((L, EMB, FF), lambda i: (0, 0, 0)),
            pl.BlockSpec((L, FF, EMB), lambda i: (0, 0, 0)),
            pl.BlockSpec((EMB, C), lambda i: (0, 0)),
            pl.BlockSpec((1, C), lambda i: (0, 0)),
        ],
        out_specs=pl.BlockSpec((BB, C), lambda i: (i, 0)),
        out_shape=jax.ShapeDtypeStruct((B, C), f32),
        scratch_shapes=[
            pltpu.VMEM((R, EMB), f32),
            pltpu.VMEM((R, EMB), f32),
            pltpu.VMEM((R, EMB), f32),
            pltpu.VMEM((R, EMB), f32),
            pltpu.VMEM((BB, EMB), f32),
            pltpu.VMEM((BB, EMB), f32),
        ],
        compiler_params=pltpu.CompilerParams(
            dimension_semantics=("parallel",)),
    )(px, wq, wk, wv, wo, w1, w2, wd, bd2)


def kernel(nodes, neigh, feat, lap, W_in, Wq, Wk, Wv, Wo, W1, W2,
           W_dense, b_dense):
    nodes32 = nodes.astype(jnp.int32)
    neigh32 = neigh.astype(jnp.int32)

    neigh_p = neigh32.reshape(N // 8, 8 * S)
    nbrows = _sc_gather_nbrows(neigh_p, (nodes32 // 8).reshape(1, B))
    j = nodes32[:, None] % 8
    nb = nbrows[:, 0:S]
    for jj in range(1, 8):
        nb = jnp.where(j == jj, nbrows[:, jj * S:(jj + 1) * S], nb)
    tok = jnp.concatenate([nodes32[:, None], nb], axis=1)
    tok2d = tok.reshape(1, BT)

    xin = _proj_kernel(feat, lap, W_in)
    px = _sc_gather_xin(xin, tok2d)

    return _tc_transformer(px, Wq, Wk, Wv, Wo, W1, W2, W_dense, b_dense)
